# Initial kernel scaffold; baseline (speedup 1.0000x reference)
import jax
import jax.numpy as jnp
from jax.experimental import pallas as pl


def kernel(x, W):
    raise NotImplementedError("write your pallas kernel here")


# speedup vs baseline: 1.9470x; 1.9470x over previous
import functools

import jax
import jax.numpy as jnp
from jax.experimental import pallas as pl
from jax.experimental.pallas import tpu as pltpu

_D_MODEL = 2048
_N_EXPERTS = 16
_TOP_K = 2
_LB_COEF = 0.01
_N_TOKENS = 16384

_BLOCK = 2048


def _router_kernel(x_ref, w_ref, rw_ref, idx_ref, loss_ref, psum_ref, cnt_ref):
    i = pl.program_id(0)
    nsteps = pl.num_programs(0)

    @pl.when(i == 0)
    def _init():
        psum_ref[...] = jnp.zeros_like(psum_ref)
        cnt_ref[...] = jnp.zeros_like(cnt_ref)

    xb = x_ref[...]
    w = w_ref[...]
    logits = jax.lax.dot_general(
        xb, w, (((1,), (1,)), ((), ())),
        preferred_element_type=jnp.float32)

    m = jnp.max(logits, axis=-1, keepdims=True)
    e = jnp.exp(logits - m)
    s = jnp.sum(e, axis=-1, keepdims=True)
    probs = e / s

    cols = jax.lax.broadcasted_iota(jnp.int32, logits.shape, 1)
    i1 = jnp.argmax(logits, axis=-1)
    top1_mask = cols == i1[:, None]
    masked = jnp.where(top1_mask, -jnp.inf, logits)
    i2 = jnp.argmax(masked, axis=-1)

    p1 = jnp.max(probs, axis=-1)
    p2 = jnp.max(jnp.where(top1_mask, -1.0, probs), axis=-1)
    denom = p1 + p2
    rw_ref[...] = jnp.stack([p1 / denom, p2 / denom], axis=1)
    idx_ref[...] = jnp.stack([i1, i2], axis=1).astype(jnp.int32)

    psum_ref[...] += jnp.sum(probs, axis=0)[None, :]
    cnt_ref[...] += jnp.sum(top1_mask.astype(jnp.float32), axis=0)[None, :]

    @pl.when(i == nsteps - 1)
    def _fin():
        n = jnp.float32(nsteps * xb.shape[0])
        f = cnt_ref[...] / n
        p = psum_ref[...] / n
        loss_ref[...] = (_LB_COEF * jnp.sum(f * p)).reshape(1, 1)


@functools.partial(jax.jit, static_argnames=())
def kernel(x, W):
    n = x.shape[0]
    grid = (n // _BLOCK,)
    rw, idx, loss = pl.pallas_call(
        _router_kernel,
        grid=grid,
        in_specs=[
            pl.BlockSpec((_BLOCK, _D_MODEL), lambda i: (i, 0)),
            pl.BlockSpec((_N_EXPERTS, _D_MODEL), lambda i: (0, 0)),
        ],
        out_specs=[
            pl.BlockSpec((_BLOCK, _TOP_K), lambda i: (i, 0)),
            pl.BlockSpec((_BLOCK, _TOP_K), lambda i: (i, 0)),
            pl.BlockSpec((1, 1), lambda i: (0, 0)),
        ],
        out_shape=[
            jax.ShapeDtypeStruct((n, _TOP_K), jnp.float32),
            jax.ShapeDtypeStruct((n, _TOP_K), jnp.int32),
            jax.ShapeDtypeStruct((1, 1), jnp.float32),
        ],
        scratch_shapes=[
            pltpu.VMEM((1, _N_EXPERTS), jnp.float32),
            pltpu.VMEM((1, _N_EXPERTS), jnp.float32),
        ],
        compiler_params=pltpu.CompilerParams(
            dimension_semantics=("arbitrary",),
        ),
    )(x, W)
    return rw, idx, loss.reshape(())
